# baseline (speedup 1.0000x reference)
import functools

import jax
import jax.numpy as jnp
from jax.experimental import pallas as pl
from jax.experimental.pallas import tpu as pltpu

_N = 1024
_D = 32
_OPTIONS = 512
_NUM_STAGES = 4
_CHAINS = 2


def _encoder_body(x_ref, cbt_ref, bias_ref, enc_ref, cur_ref, loss_hbm,
                  loss_scr, sems):
    n = _N // _CHAINS
    ones_col = jnp.ones((n, 1), jnp.float32)
    ones_row = jnp.ones((1, _OPTIONS), jnp.float32)
    xs = [x_ref[pl.ds(c * n, n), :] for c in range(_CHAINS)]
    currents = [jnp.zeros_like(xs[c]) for c in range(_CHAINS)]
    idxs = [[] for _ in range(_CHAINS)]
    copies = []
    b_augs = []
    for i in range(_NUM_STAGES):
        cbt = cbt_ref[i]
        if i == 0:
            cbt = cbt + bias_ref[...].T
        sq_c = jnp.sum(cbt * cbt, axis=0, keepdims=True) * (1.0 / _D)
        b_augs.append((cbt, jnp.concatenate([cbt, ones_row, sq_c], axis=0)))

    def stage(c, i):
        r = currents[c] - xs[c]
        sq_r = jnp.sum(r * r, axis=1, keepdims=True) * (1.0 / _D)
        a_aug = jnp.concatenate([r * (2.0 / _D), sq_r, ones_col], axis=1)
        cbt, b_aug = b_augs[i]
        loss = jax.lax.dot_general(
            a_aug, b_aug, (((1,), (0,)), ((), ())),
            preferred_element_type=jnp.float32,
            precision=jax.lax.Precision.HIGHEST,
        )
        loss_scr[i, pl.ds(c * n, n), :] = loss
        cp = pltpu.make_async_copy(
            loss_scr.at[i, pl.ds(c * n, n), :],
            loss_hbm.at[pl.ds(c * n, n), i, :],
            sems.at[i * _CHAINS + c],
        )
        cp.start()
        copies.append(cp)
        idx = jnp.argmin(loss, axis=1)
        idxs[c].append(idx)
        onehot_t = (
            jax.lax.broadcasted_iota(jnp.int32, (_OPTIONS, n), 0)
            == idx[None, :]
        ).astype(jnp.float32)
        chosen_t = jax.lax.dot_general(
            cbt, onehot_t, (((1,), (0,)), ((), ())),
            preferred_element_type=jnp.float32,
            precision=jax.lax.Precision.HIGHEST,
        )
        currents[c] = currents[c] + chosen_t.T

    for i in range(_NUM_STAGES):
        for c in range(_CHAINS):
            stage(c, i)
    enc_ref[...] = jnp.concatenate(
        [jnp.stack(ix, axis=1) for ix in idxs], axis=0
    )
    cur_ref[...] = jnp.concatenate(currents, axis=0)
    for cp in copies:
        cp.wait()


@jax.jit
def kernel(inputs, codebook, bias):
    n, d = inputs.shape
    num_stages, options, _ = codebook.shape
    cbt = jnp.swapaxes(codebook, 1, 2)
    enc, current, losses = pl.pallas_call(
        _encoder_body,
        in_specs=[
            pl.BlockSpec((n, d), lambda: (0, 0)),
            pl.BlockSpec((num_stages, d, options), lambda: (0, 0, 0)),
            pl.BlockSpec((options, d), lambda: (0, 0)),
        ],
        out_specs=[
            pl.BlockSpec((n, num_stages), lambda: (0, 0)),
            pl.BlockSpec((n, d), lambda: (0, 0)),
            pl.BlockSpec(memory_space=pltpu.MemorySpace.HBM),
        ],
        out_shape=[
            jax.ShapeDtypeStruct((n, num_stages), jnp.int32),
            jax.ShapeDtypeStruct((n, d), jnp.float32),
            jax.ShapeDtypeStruct((n, num_stages, options), jnp.float32),
        ],
        scratch_shapes=[
            pltpu.VMEM((num_stages, n, options), jnp.float32),
            pltpu.SemaphoreType.DMA((num_stages * _CHAINS,)),
        ],
    )(inputs, cbt, bias)
    return enc, current, losses
